# Initial kernel scaffold; baseline (speedup 1.0000x reference)
import jax
import jax.numpy as jnp
from jax.experimental import pallas as pl


def kernel(x, edge_index, W1, b1, W4, b4):
    raise NotImplementedError("write your pallas kernel here")


# speedup vs baseline: 21.7483x; 21.7483x over previous
import functools

import jax
import jax.numpy as jnp
from jax import lax
from jax.experimental import pallas as pl
from jax.experimental.pallas import tpu as pltpu
from jax.experimental.pallas import tpu_sc as plsc

NC = 2
NS = 16
NW = NC * NS
LANES = 16
CH = 128

_f32 = jnp.float32
_i32 = jnp.int32


def _mesh():
    return plsc.VectorSubcoreMesh(core_axis_name="c", subcore_axis_name="s")


_SC_PARAMS = pltpu.CompilerParams(needs_layout_passes=False)


def _wid():
    return lax.axis_index("s") * NC + lax.axis_index("c")


def _deg_body(n_pad, epw, dst_hbm, out_hbm, dstbuf, acc):
    wid = _wid()

    def zero(i, _):
        acc[pl.ds(i * LANES, LANES)] = jnp.zeros((LANES,), _f32)
        return 0

    lax.fori_loop(0, n_pad // LANES, zero, 0)
    pltpu.sync_copy(dst_hbm.at[pl.ds(wid * epw, epw)], dstbuf)
    ones = jnp.ones((LANES,), _f32)

    def body(i, _):
        idx = dstbuf[pl.ds(i * LANES, LANES)]
        plsc.addupdate_scatter(acc, [idx], ones)
        return 0

    lax.fori_loop(0, epw // LANES, body, 0)
    pltpu.sync_copy(acc, out_hbm.at[wid])


def _sc_degree(dst_p, n_pad):
    epw = dst_p.shape[0] // NW
    k = functools.partial(
        pl.kernel,
        out_type=jax.ShapeDtypeStruct((NW, n_pad), _f32),
        mesh=_mesh(),
        compiler_params=_SC_PARAMS,
        scratch_types=[
            pltpu.VMEM((epw,), _i32),
            pltpu.VMEM((n_pad,), _f32),
        ],
    )(functools.partial(_deg_body, n_pad, epw))
    return k(dst_p)


def _agg_body(n_pad, cpw, hp_hbm, src_hbm, dst_hbm, zeros_hbm, out_hbm,
              srcv, dstv, rows, acc_sh, sem):
    c = lax.axis_index("c")
    s = lax.axis_index("s")
    wid = s * NC + c
    rpt = n_pad // NS
    pltpu.sync_copy(zeros_hbm, acc_sh.at[pl.ds(s * rpt, rpt)])
    plsc.subcore_barrier()

    def body(i, _):
        base = (wid * cpw + i) * CH
        pltpu.sync_copy(src_hbm.at[pl.ds(base, CH)], srcv)
        pltpu.sync_copy(dst_hbm.at[pl.ds(base, CH)], dstv)
        pltpu.async_copy(hp_hbm.at[srcv], rows, sem).wait()
        pltpu.sync_copy(rows, acc_sh.at[dstv], add=True)
        return 0

    lax.fori_loop(0, cpw, body, 0)
    plsc.subcore_barrier()
    pltpu.sync_copy(acc_sh.at[pl.ds(s * rpt, rpt)],
                    out_hbm.at[c, pl.ds(s * rpt, rpt)])


def _sc_aggregate(hp, src_p, dst_p, n_pad):
    d = hp.shape[1]
    cpw = src_p.shape[0] // (NW * CH)
    rpt = n_pad // NS
    zeros = jnp.zeros((rpt, d), _f32)
    k = functools.partial(
        pl.kernel,
        out_type=jax.ShapeDtypeStruct((NC, n_pad, d), _f32),
        mesh=_mesh(),
        compiler_params=_SC_PARAMS,
        scratch_types=[
            pltpu.VMEM((CH,), _i32),
            pltpu.VMEM((CH,), _i32),
            pltpu.VMEM((CH, d), _f32),
            pltpu.VMEM_SHARED((n_pad, d), _f32),
            pltpu.SemaphoreType.DMA,
        ],
    )(functools.partial(_agg_body, n_pad, cpw))
    return k(hp, src_p, dst_p, zeros)


def _l2_body(n, n_pad, epw, z_hbm, src_hbm, dst_hbm, out_hbm,
             ztab, srcbuf, dstbuf, acc):
    wid = _wid()

    def zero(i, _):
        acc[pl.ds(i * LANES, LANES)] = jnp.zeros((LANES,), _f32)
        return 0

    lax.fori_loop(0, n_pad // LANES, zero, 0)
    pltpu.sync_copy(z_hbm, ztab)
    pltpu.sync_copy(src_hbm.at[pl.ds(wid * epw, epw)], srcbuf)
    pltpu.sync_copy(dst_hbm.at[pl.ds(wid * epw, epw)], dstbuf)

    def body(i, _):
        si = srcbuf[pl.ds(i * LANES, LANES)]
        vals = plsc.load_gather(ztab, [si])
        di = dstbuf[pl.ds(i * LANES, LANES)]
        plsc.addupdate_scatter(acc, [di], vals)
        return 0

    lax.fori_loop(0, epw // LANES, body, 0)
    pltpu.sync_copy(acc, out_hbm.at[wid])


def _sc_l2(z, src_p, dst_p, n_pad):
    n = z.shape[0]
    epw = src_p.shape[0] // NW
    k = functools.partial(
        pl.kernel,
        out_type=jax.ShapeDtypeStruct((NW, n_pad), _f32),
        mesh=_mesh(),
        compiler_params=_SC_PARAMS,
        scratch_types=[
            pltpu.VMEM((n,), _f32),
            pltpu.VMEM((epw,), _i32),
            pltpu.VMEM((epw,), _i32),
            pltpu.VMEM((n_pad,), _f32),
        ],
    )(functools.partial(_l2_body, n, n_pad, epw))
    return k(z, src_p, dst_p)


def _tc1_body(n, x_ref, w1_ref, degp_ref, hp_ref, dinv_ref):
    deg = jnp.sum(degp_ref[...], axis=1, keepdims=True)[:n, :] + 1.0
    dinv = lax.rsqrt(deg)
    h = jnp.dot(x_ref[...], w1_ref[...], preferred_element_type=_f32)
    hp_ref[...] = h * dinv
    dinv_ref[...] = dinv


def _tc2_body(n, agg_ref, hp_ref, dinv_ref, b1_ref, w4_ref, zp_ref):
    ssum = agg_ref[0, :n, :] + agg_ref[1, :n, :] + hp_ref[...]
    h1 = jnp.maximum(ssum * dinv_ref[...] + b1_ref[...], 0.0)
    z = jnp.dot(h1, w4_ref[...], preferred_element_type=_f32)
    zp_ref[...] = z * dinv_ref[...]


def _tc3_body(n, l2p_ref, zp_ref, dinv_ref, b4_ref, out_ref):
    t = jnp.sum(l2p_ref[...], axis=1, keepdims=True)[:n, :] + zp_ref[...]
    out_ref[...] = jax.nn.sigmoid(t * dinv_ref[...] + b4_ref[...])


def kernel(x, edge_index, W1, b1, W4, b4):
    n, nf = x.shape
    hid = W1.shape[1]
    e = edge_index.shape[1]
    n_pad = ((n + NS * LANES) // (NS * LANES)) * NS * LANES
    cpw = -(-e // (NW * CH))
    e_pad = NW * CH * cpw
    pad = e_pad - e

    src = edge_index[0].astype(_i32)
    dst = edge_index[1].astype(_i32)
    src_p = jnp.concatenate([src, jnp.zeros((pad,), _i32)])
    dst_p = jnp.concatenate([dst, jnp.full((pad,), n, _i32)])

    deg_parts = _sc_degree(dst_p, n_pad).T

    hp, dinv = pl.pallas_call(
        functools.partial(_tc1_body, n),
        out_shape=(jax.ShapeDtypeStruct((n, hid), _f32),
                   jax.ShapeDtypeStruct((n, 1), _f32)),
    )(x, W1, deg_parts)

    agg = _sc_aggregate(hp, src_p, dst_p, n_pad)

    zp = pl.pallas_call(
        functools.partial(_tc2_body, n),
        out_shape=jax.ShapeDtypeStruct((n, 1), _f32),
    )(agg, hp, dinv, b1, W4)

    l2_parts = _sc_l2(zp.reshape(n), src_p, dst_p, n_pad).T

    out = pl.pallas_call(
        functools.partial(_tc3_body, n),
        out_shape=jax.ShapeDtypeStruct((n, 1), _f32),
    )(l2_parts, zp, dinv, b4)
    return out
